# baseline (speedup 1.0000x reference)
import jax, jax.numpy as jnp
from jax import lax
from jax.experimental import pallas as pl
from jax.experimental.pallas import tpu as pltpu

NV_PREV = 10242
BLK = 1664

def _body(x_ref, o_ref):
    o_ref[...] = x_ref[...]

def kernel(x):
    n, a, b, c = x.shape
    xt = lax.transpose(x, (1, 2, 3, 0))
    out_t = pl.pallas_call(
        _body,
        grid=(pl.cdiv(NV_PREV, BLK),),
        in_specs=[pl.BlockSpec((a, b, c, BLK), lambda i: (0, 0, 0, i))],
        out_specs=pl.BlockSpec((a, b, c, BLK), lambda i: (0, 0, 0, i)),
        out_shape=jax.ShapeDtypeStruct((a, b, c, NV_PREV), x.dtype),
    )(xt)
    return lax.transpose(out_t, (3, 0, 1, 2))


# speedup vs baseline: 1.0116x; 1.0116x over previous
import jax, jax.numpy as jnp
from jax import lax
from jax.experimental import pallas as pl
from jax.experimental.pallas import tpu as pltpu

NV_PREV = 10242
BLK = 1920

def _body(x_ref, o_ref):
    o_ref[...] = x_ref[...]

def kernel(x):
    n, a, b, c = x.shape
    xt = lax.transpose(x, (1, 2, 3, 0))
    out_t = pl.pallas_call(
        _body,
        grid=(pl.cdiv(NV_PREV, BLK),),
        in_specs=[pl.BlockSpec((a, b, c, BLK), lambda i: (0, 0, 0, i))],
        out_specs=pl.BlockSpec((a, b, c, BLK), lambda i: (0, 0, 0, i)),
        out_shape=jax.ShapeDtypeStruct((a, b, c, NV_PREV), x.dtype),
    )(xt)
    return lax.transpose(out_t, (3, 0, 1, 2))


# speedup vs baseline: 1.0416x; 1.0297x over previous
import jax, jax.numpy as jnp
from jax import lax
from jax.experimental import pallas as pl
from jax.experimental.pallas import tpu as pltpu

NV_PREV = 10242
BLK = 1792

def _body(x_ref, o_ref):
    o_ref[...] = x_ref[...]

def kernel(x):
    n, a, b, c = x.shape
    xt = lax.transpose(x, (1, 2, 3, 0))
    out_t = pl.pallas_call(
        _body,
        grid=(pl.cdiv(NV_PREV, BLK),),
        in_specs=[pl.BlockSpec((a, b, c, BLK), lambda i: (0, 0, 0, i))],
        out_specs=pl.BlockSpec((a, b, c, BLK), lambda i: (0, 0, 0, i)),
        out_shape=jax.ShapeDtypeStruct((a, b, c, NV_PREV), x.dtype),
    )(xt)
    return lax.transpose(out_t, (3, 0, 1, 2))
